# baseline (speedup 1.0000x reference)
import functools

import jax
import jax.numpy as jnp
from jax import lax
from jax.experimental import pallas as pl
from jax.experimental.pallas import tpu as pltpu
from jax.experimental.pallas import tpu_sc as plsc

N = 10000
E = 320000
D = 128

NC = 2
NS = 16
NW = NC * NS
CH = 80
NCHUNK = 128
EPW = CH * NCHUNK
EPAD = NW * EPW
NBUF = 4
PIPE = NBUF // 2
IRING = 2 * NBUF
NACC = N + 8
RPT = N // NS

BN = 1000
NB = N // BN


def _sc_agg(interpret=False, sync_scatter=False, no_scatter=False,
            serial=False):
    mesh = plsc.VectorSubcoreMesh(core_axis_name="c", subcore_axis_name="s")

    @functools.partial(
        pl.kernel,
        out_type=jax.ShapeDtypeStruct((NC, NS, RPT, D), jnp.float32),
        mesh=mesh,
        scratch_types=[
            [pltpu.VMEM((CH,), jnp.int32) for _ in range(IRING)],
            [pltpu.VMEM((CH,), jnp.int32) for _ in range(IRING)],
            [pltpu.VMEM((CH, D), jnp.float32) for _ in range(NBUF)],
            pltpu.VMEM_SHARED((NACC, D), jnp.float32),
            [pltpu.SemaphoreType.DMA for _ in range(IRING)],
            [pltpu.SemaphoreType.DMA for _ in range(IRING)],
            [pltpu.SemaphoreType.DMA for _ in range(NBUF)],
            [pltpu.SemaphoreType.DMA for _ in range(NBUF)],
        ],
        interpret=interpret,
    )
    def agg(h_hbm, src_hbm, dst_hbm, zeros_hbm, out_hbm,
            src1, dst1, rows, acc_sh, s_sem, d_sem, gsem, ssem):
        cid = lax.axis_index("c")
        sid = lax.axis_index("s")
        wid = sid * NC + cid
        base = wid * EPW

        def load_idx(j, r):
            off = base + j * CH
            pltpu.async_copy(src_hbm.at[pl.ds(off, CH)], src1[r], s_sem[r])
            pltpu.async_copy(dst_hbm.at[pl.ds(off, CH)], dst1[r], d_sem[r])

        def start_gather(j_r, b):
            pltpu.make_async_copy(src_hbm.at[pl.ds(0, CH)], src1[j_r],
                                  s_sem[j_r]).wait()
            pltpu.async_copy(h_hbm.at[src1[j_r]], rows[b], gsem[b])

        def wait_gather(b):
            pltpu.make_async_copy(h_hbm.at[src1[0]], rows[b], gsem[b]).wait()

        def start_scatter(j_r, b):
            pltpu.make_async_copy(dst_hbm.at[pl.ds(0, CH)], dst1[j_r],
                                  d_sem[j_r]).wait()
            pltpu.async_copy(rows[b], acc_sh.at[dst1[j_r]], ssem[b], add=True)

        def wait_scatter(b):
            pltpu.make_async_copy(rows[b], acc_sh.at[dst1[0]],
                                  ssem[b]).wait()

        if serial:
            pltpu.sync_copy(zeros_hbm, acc_sh.at[pl.ds(sid * RPT, RPT)])
            plsc.subcore_barrier()

            def sbody(j, carry):
                load_idx(j, 0)
                start_gather(0, 0)
                wait_gather(0)
                start_scatter(0, 0)
                wait_scatter(0)
                return carry

            lax.fori_loop(0, NCHUNK, sbody, 0)
            plsc.subcore_barrier()
            pltpu.sync_copy(acc_sh.at[pl.ds(sid * RPT, RPT)],
                            out_hbm.at[cid, sid])
            return

        for r in range(2 * PIPE):
            load_idx(r, r)
        pltpu.sync_copy(zeros_hbm, acc_sh.at[pl.ds(sid * RPT, RPT)])
        for b in range(PIPE):
            start_gather(b, b)
        plsc.subcore_barrier()

        def outer(o, carry):
            for k in range(IRING):
                j = o * IRING + k
                b = k % NBUF
                bg = (k + PIPE) % NBUF

                if not no_scatter:
                    @pl.when(j >= PIPE)
                    def _():
                        wait_scatter(bg)

                @pl.when(j + 2 * PIPE < NCHUNK)
                def _():
                    load_idx(j + 2 * PIPE, (k + 2 * PIPE) % IRING)

                @pl.when(j + PIPE < NCHUNK)
                def _():
                    start_gather((k + PIPE) % IRING, bg)

                wait_gather(b)
                if not no_scatter:
                    start_scatter(k, b)
                    if sync_scatter:
                        wait_scatter(b)
            return carry

        lax.fori_loop(0, NCHUNK // IRING, outer, 0)
        if not (no_scatter or sync_scatter):
            for i in range(PIPE):
                wait_scatter((NCHUNK - PIPE + i) % NBUF)
        plsc.subcore_barrier()

        pltpu.sync_copy(acc_sh.at[pl.ds(sid * RPT, RPT)], out_hbm.at[cid, sid])

    return agg


_AGG = _sc_agg(serial=True)


def _mlp_body(eps_ref, x_ref, p0_ref, p1_ref, w1_ref, b1_ref, w2_ref, b2_ref,
              o_ref):
    u = (1.0 + eps_ref[0, 0]) * x_ref[...] + p0_ref[...] + p1_ref[...]
    h1 = jnp.dot(u, w1_ref[...], preferred_element_type=jnp.float32)
    h1 = jnp.maximum(h1 + b1_ref[...], 0.0)
    h2 = jnp.dot(h1, w2_ref[...], preferred_element_type=jnp.float32)
    o_ref[...] = jnp.maximum(h2 + b2_ref[...], 0.0)


def _final_body(eps_ref, bf2_ref, x_ref, p0_ref, p1_ref, w1_ref, b1_ref,
                w2_ref, b2_ref, wf1_ref, bf1_ref, wf2_ref, o_ref):
    u = (1.0 + eps_ref[0, 0]) * x_ref[...] + p0_ref[...] + p1_ref[...]
    h1 = jnp.dot(u, w1_ref[...], preferred_element_type=jnp.float32)
    h1 = jnp.maximum(h1 + b1_ref[...], 0.0)
    h2 = jnp.dot(h1, w2_ref[...], preferred_element_type=jnp.float32)
    h2 = jnp.maximum(h2 + b2_ref[...], 0.0)
    h3 = jnp.dot(h2, wf1_ref[...], preferred_element_type=jnp.float32)
    h3 = jnp.maximum(h3 + bf1_ref[...], 0.0)
    s = jnp.sum(h3 * wf2_ref[...], axis=1, keepdims=True) + bf2_ref[0, 0]
    o_ref[...] = 1.0 / (1.0 + jnp.exp(-s))


_ROWS = pl.BlockSpec((BN, D), lambda i: (i, 0))
_P0 = pl.BlockSpec((BN, D), lambda i: (i, 0))
_P1 = pl.BlockSpec((BN, D), lambda i: (i + NB, 0))
_WMAT = pl.BlockSpec((D, D), lambda i: (0, 0))
_BVEC = pl.BlockSpec((1, D), lambda i: (0, 0))
_SCALAR = pl.BlockSpec(memory_space=pltpu.SMEM)

_mlp_call = pl.pallas_call(
    _mlp_body,
    grid=(NB,),
    in_specs=[_SCALAR, _ROWS, _P0, _P1, _WMAT, _BVEC, _WMAT, _BVEC],
    out_specs=_ROWS,
    out_shape=jax.ShapeDtypeStruct((N, D), jnp.float32),
)

_final_call = pl.pallas_call(
    _final_body,
    grid=(NB,),
    in_specs=[_SCALAR, _SCALAR, _ROWS, _P0, _P1, _WMAT, _BVEC, _WMAT, _BVEC,
              pl.BlockSpec((D, D // 2), lambda i: (0, 0)),
              pl.BlockSpec((1, D // 2), lambda i: (0, 0)),
              pl.BlockSpec((1, D // 2), lambda i: (0, 0))],
    out_specs=pl.BlockSpec((BN, 1), lambda i: (i, 0)),
    out_shape=jax.ShapeDtypeStruct((N, 1), jnp.float32),
)


def kernel(x, edge_index, W1_0, b1_0, W2_0, b2_0, eps0, W1_1, b1_1, W2_1,
           b2_1, eps1, W1_2, b1_2, W2_2, b2_2, eps2, Wf1, bf1, Wf2, bf2):
    pad = EPAD - E
    src = jnp.concatenate([edge_index[0], jnp.zeros((pad,), jnp.int32)])
    dst = jnp.concatenate([edge_index[1], jnp.full((pad,), N, jnp.int32)])
    zeros = jnp.zeros((RPT, D), jnp.float32)

    h = x
    for eps, W1, b1, W2, b2 in ((eps0, W1_0, b1_0, W2_0, b2_0),
                                (eps1, W1_1, b1_1, W2_1, b2_1)):
        pcat = _AGG(h, src, dst, zeros).reshape(NC * N, D)
        h = _mlp_call(eps.reshape(1, 1), h, pcat, pcat, W1,
                      b1.reshape(1, D), W2, b2.reshape(1, D))

    pcat = _AGG(h, src, dst, zeros).reshape(NC * N, D)
    return _final_call(eps2.reshape(1, 1), bf2.reshape(1, 1), h, pcat, pcat,
                       W1_2, b1_2.reshape(1, D), W2_2, b2_2.reshape(1, D),
                       Wf1, bf1.reshape(1, D // 2), Wf2.reshape(1, D // 2))


# speedup vs baseline: 1.7372x; 1.7372x over previous
import functools

import jax
import jax.numpy as jnp
from jax import lax
from jax.experimental import pallas as pl
from jax.experimental.pallas import tpu as pltpu
from jax.experimental.pallas import tpu_sc as plsc

N = 10000
E = 320000
D = 128

NC = 2
NS = 16
NW = NC * NS
EPW = E // NW
CH = 80
NCHUNK = EPW // CH
RPT = N // NS
ZR = 125

BN = 1000
NB = N // BN


def _sc_agg():
    mesh = plsc.VectorSubcoreMesh(core_axis_name="c", subcore_axis_name="s")

    @functools.partial(
        pl.kernel,
        out_type=jax.ShapeDtypeStruct((NC, NS, RPT, D), jnp.float32),
        mesh=mesh,
        scratch_types=[
            pltpu.VMEM((CH,), jnp.int32),
            pltpu.VMEM((CH,), jnp.int32),
            pltpu.VMEM((CH, D), jnp.float32),
            pltpu.VMEM((ZR, D), jnp.float32),
            pltpu.VMEM_SHARED((N, D), jnp.float32),
            pltpu.SemaphoreType.DMA,
        ],
    )
    def agg(h_hbm, src_hbm, dst_hbm, zeros_hbm, out_hbm,
            src_v, dst_v, rows_v, zb_v, acc_sh, sem):
        cid = lax.axis_index("c")
        sid = lax.axis_index("s")
        wid = sid * NC + cid

        pltpu.sync_copy(zeros_hbm, zb_v)
        for k in range(RPT // ZR):
            pltpu.sync_copy(zb_v, acc_sh.at[pl.ds(sid * RPT + k * ZR, ZR)])
        plsc.subcore_barrier()

        base = wid * EPW

        def body(i, carry):
            off = base + i * CH
            pltpu.sync_copy(src_hbm.at[pl.ds(off, CH)], src_v)
            pltpu.sync_copy(dst_hbm.at[pl.ds(off, CH)], dst_v)
            pltpu.async_copy(h_hbm.at[src_v], rows_v, sem).wait()
            pltpu.sync_copy(rows_v, acc_sh.at[dst_v], add=True)
            return carry

        lax.fori_loop(0, NCHUNK, body, 0)
        plsc.subcore_barrier()

        pltpu.sync_copy(acc_sh.at[pl.ds(sid * RPT, RPT)], out_hbm.at[cid, sid])

    return agg


_AGG = _sc_agg()


def _mlp_body(eps_ref, x_ref, p0_ref, p1_ref, w1_ref, b1_ref, w2_ref, b2_ref,
              o_ref):
    u = (1.0 + eps_ref[0, 0]) * x_ref[...] + p0_ref[...] + p1_ref[...]
    h1 = jnp.dot(u, w1_ref[...], preferred_element_type=jnp.float32)
    h1 = jnp.maximum(h1 + b1_ref[...], 0.0)
    h2 = jnp.dot(h1, w2_ref[...], preferred_element_type=jnp.float32)
    o_ref[...] = jnp.maximum(h2 + b2_ref[...], 0.0)


def _final_body(eps_ref, bf2_ref, x_ref, p0_ref, p1_ref, w1_ref, b1_ref,
                w2_ref, b2_ref, wf1_ref, bf1_ref, wf2_ref, o_ref):
    u = (1.0 + eps_ref[0, 0]) * x_ref[...] + p0_ref[...] + p1_ref[...]
    h1 = jnp.dot(u, w1_ref[...], preferred_element_type=jnp.float32)
    h1 = jnp.maximum(h1 + b1_ref[...], 0.0)
    h2 = jnp.dot(h1, w2_ref[...], preferred_element_type=jnp.float32)
    h2 = jnp.maximum(h2 + b2_ref[...], 0.0)
    h3 = jnp.dot(h2, wf1_ref[...], preferred_element_type=jnp.float32)
    h3 = jnp.maximum(h3 + bf1_ref[...], 0.0)
    s = jnp.sum(h3 * wf2_ref[...], axis=1, keepdims=True) + bf2_ref[0, 0]
    o_ref[...] = 1.0 / (1.0 + jnp.exp(-s))


_ROWS = pl.BlockSpec((BN, D), lambda i: (i, 0))
_P0 = pl.BlockSpec((BN, D), lambda i: (i, 0))
_P1 = pl.BlockSpec((BN, D), lambda i: (i + NB, 0))
_WMAT = pl.BlockSpec((D, D), lambda i: (0, 0))
_BVEC = pl.BlockSpec((1, D), lambda i: (0, 0))
_SCALAR = pl.BlockSpec(memory_space=pltpu.SMEM)

_mlp_call = pl.pallas_call(
    _mlp_body,
    grid=(NB,),
    in_specs=[_SCALAR, _ROWS, _P0, _P1, _WMAT, _BVEC, _WMAT, _BVEC],
    out_specs=_ROWS,
    out_shape=jax.ShapeDtypeStruct((N, D), jnp.float32),
)

_final_call = pl.pallas_call(
    _final_body,
    grid=(NB,),
    in_specs=[_SCALAR, _SCALAR, _ROWS, _P0, _P1, _WMAT, _BVEC, _WMAT, _BVEC,
              pl.BlockSpec((D, D // 2), lambda i: (0, 0)),
              pl.BlockSpec((1, D // 2), lambda i: (0, 0)),
              pl.BlockSpec((1, D // 2), lambda i: (0, 0))],
    out_specs=pl.BlockSpec((BN, 1), lambda i: (i, 0)),
    out_shape=jax.ShapeDtypeStruct((N, 1), jnp.float32),
)


def kernel(x, edge_index, W1_0, b1_0, W2_0, b2_0, eps0, W1_1, b1_1, W2_1,
           b2_1, eps1, W1_2, b1_2, W2_2, b2_2, eps2, Wf1, bf1, Wf2, bf2):
    src = edge_index[0]
    dst = edge_index[1]
    zeros = jnp.zeros((ZR, D), jnp.float32)

    h = x
    for eps, W1, b1, W2, b2 in ((eps0, W1_0, b1_0, W2_0, b2_0),
                                (eps1, W1_1, b1_1, W2_1, b2_1)):
        pcat = _AGG(h, src, dst, zeros).reshape(NC * N, D)
        h = _mlp_call(eps.reshape(1, 1), h, pcat, pcat, W1,
                      b1.reshape(1, D), W2, b2.reshape(1, D))

    pcat = _AGG(h, src, dst, zeros).reshape(NC * N, D)
    return _final_call(eps2.reshape(1, 1), bf2.reshape(1, 1), h, pcat, pcat,
                       W1_2, b1_2.reshape(1, D), W2_2, b2_2.reshape(1, D),
                       Wf1, bf1.reshape(1, D // 2), Wf2.reshape(1, D // 2))
